# baseline (speedup 1.0000x reference)
import jax
import jax.numpy as jnp
from jax.experimental import pallas as pl
from jax.experimental.pallas import tpu as pltpu

_MIN_CLASSES = (3, 17, 42, 101, 256, 511, 640, 777, 888, 999)
_LOSS_WEIGHT = 1.0


def _tc_body(x_ref, lab_ref, out_ref):
    i = pl.program_id(0)
    x = x_ref[...]
    lab = lab_ref[0, 0, :]
    m = jnp.max(x, axis=1, keepdims=True)
    s = jnp.sum(jnp.exp(x - m), axis=1)
    lse = m[:, 0] + jnp.log(s)

    keep = jnp.ones_like(lab, dtype=jnp.bool_)
    for c in _MIN_CLASSES:
        keep = jnp.logical_and(keep, lab != c)
    keep_f = keep.astype(jnp.float32)

    col = jax.lax.broadcasted_iota(jnp.int32, x.shape, 1)
    xg = jnp.sum(jnp.where(col == lab[:, None], x, 0.0), axis=1)

    part_nll = jnp.sum(keep_f * (lse - xg))
    part_cnt = jnp.sum(keep_f)

    @pl.when(i == 0)
    def _init():
        out_ref[0, 0] = 0.0
        out_ref[0, 1] = 0.0

    out_ref[0, 0] += part_nll
    out_ref[0, 1] += part_cnt


def kernel(cls_score, label):
    n, c = cls_score.shape
    block = 512
    grid = n // block
    lab3 = label.astype(jnp.int32).reshape(grid, 1, block)
    sums = pl.pallas_call(
        _tc_body,
        grid=(grid,),
        in_specs=[
            pl.BlockSpec((block, c), lambda i: (i, 0)),
            pl.BlockSpec((1, 1, block), lambda i: (i, 0, 0)),
        ],
        out_specs=pl.BlockSpec(memory_space=pltpu.SMEM),
        out_shape=jax.ShapeDtypeStruct((1, 2), jnp.float32),
    )(cls_score, lab3)
    loss = sums[0, 0] / jnp.maximum(sums[0, 1], 1.0)
    return _LOSS_WEIGHT * loss
